# baseline (speedup 1.0000x reference)
import functools

import jax
import jax.numpy as jnp
from jax import lax
from jax.experimental import pallas as pl
from jax.experimental.pallas import tpu as pltpu
from jax.experimental.pallas import tpu_sc as plsc

BATCH = 4096
SEQ = 200
EMBED_DIM = 64
PAD_DIM = 128
LANES = 16
NUM_WORKERS = 32
ROWS_PER_W = BATCH // NUM_WORKERS
CHUNK = 48
LAST_OFF = SEQ - CHUNK
NBUF = 2

VOCAB = 1000000
VBLK = 128
NBLK = (VOCAB + VBLK - 1) // VBLK
BLK_PER_W = NBLK // NUM_WORKERS
BLK_REM = NBLK - BLK_PER_W * NUM_WORKERS


def _lin_body(tab_hbm, lin_hbm, tin_v, tout_v, sem_in, sem_out):
    cid = lax.axis_index("c")
    sid = lax.axis_index("s")
    wid = sid * 2 + cid
    start = BLK_PER_W * wid + lax.min(wid, BLK_REM)
    count = BLK_PER_W + jnp.where(wid < BLK_REM, 1, 0)

    lane_iota = lax.iota(jnp.int32, LANES)
    sems = (sem_in, sem_out)

    def rows_of(j):
        return lax.min(VBLK, VOCAB - j * VBLK)

    def fire(j, buf):
        pltpu.async_copy(
            tab_hbm.at[pl.ds(j * VBLK, rows_of(j)), :],
            tin_v.at[buf, pl.ds(0, rows_of(j)), :],
            sems[buf],
        )

    def drain_pack_store(j, buf):
        nr = rows_of(j)
        pltpu.make_async_copy(
            tab_hbm.at[pl.ds(j * VBLK, nr), :],
            tin_v.at[buf, pl.ds(0, nr), :],
            sems[buf],
        ).wait()

        def pack16(q, _):
            r0 = q * LANES
            vals = []
            for r in range(LANES):
                rvec = jnp.full((LANES,), r0 + r, jnp.int32)
                for l in range(EMBED_DIM // LANES):
                    vals.append(plsc.load_gather(
                        tin_v.at[buf], [rvec, lane_iota + l * LANES]))
            k = 0
            for r in range(LANES):
                for l in range(EMBED_DIM // LANES):
                    tout_v[buf,
                           pl.ds((r0 + r) * EMBED_DIM + l * LANES, LANES)] \
                        = vals[k]
                    k += 1
            return 0

        lax.fori_loop(0, lax.div(nr, LANES), pack16, 0)
        pltpu.sync_copy(
            tout_v.at[buf, pl.ds(0, nr * EMBED_DIM)],
            lin_hbm.at[pl.ds(j * VBLK * EMBED_DIM, nr * EMBED_DIM)],
        )

    fire(start, 0)

    def grp(g, _):
        k0 = 2 * g

        @pl.when(k0 + 1 < count)
        def _():
            fire(start + k0 + 1, 1)

        drain_pack_store(start + k0, 0)

        @pl.when(k0 + 2 < count)
        def _():
            fire(start + k0 + 2, 0)

        @pl.when(k0 + 1 < count)
        def _():
            drain_pack_store(start + k0 + 1, 1)

        return 0

    lax.fori_loop(0, (count + 1) // 2, grp, 0)


def _body(ids_hbm, lens_hbm, table_hbm, out_hbm, ids_v, lens_v, rows_v,
          out_v, sem0, sem1):
    cid = lax.axis_index("c")
    sid = lax.axis_index("s")
    wid = sid * 2 + cid
    base = wid * ROWS_PER_W
    sems = (sem0, sem1)

    pltpu.sync_copy(ids_hbm.at[pl.ds(base * SEQ, ROWS_PER_W * SEQ)], ids_v)
    pltpu.sync_copy(lens_hbm.at[pl.ds(base, ROWS_PER_W)],
                    lens_v.at[pl.ds(0, ROWS_PER_W)])

    def nchunks(b):
        ln = lens_v[pl.ds(b, LANES)][0]
        return ln, lax.div(ln + (CHUNK - 1), CHUNK)

    def fire(b, buf):
        _, nch = nchunks(b)

        def chunk(c, _):
            off = lax.min(c * CHUNK, LAST_OFF)
            pltpu.async_copy(
                table_hbm.at[ids_v.at[pl.ds(b * SEQ + off, CHUNK)]],
                rows_v.at[buf, pl.ds(off, CHUNK), :],
                sems[buf],
            )
            return 0

        lax.fori_loop(0, nch, chunk, 0)

    def drain_sum(b, buf):
        ln, nch = nchunks(b)

        def dchunk(c, _):
            off = lax.min(c * CHUNK, LAST_OFF)
            pltpu.make_async_copy(
                table_hbm.at[ids_v.at[pl.ds(b * SEQ + off, CHUNK)]],
                rows_v.at[buf, pl.ds(off, CHUNK), :],
                sems[buf],
            ).wait()
            return 0

        lax.fori_loop(0, nch, dchunk, 0)

        def load4(s):
            return [rows_v[buf, s, pl.ds(l * LANES, LANES)]
                    for l in range(4)]

        zero = jnp.zeros((LANES,), jnp.float32)

        def accum2(i, acc):
            r0 = load4(2 * i)
            r1 = load4(2 * i + 1)
            keep = jnp.full((LANES,), 2 * i + 1 < ln)
            return tuple(
                acc[l] + r0[l] + lax.select(keep, r1[l], zero)
                for l in range(4)
            )

        acc0 = tuple(zero for _ in range(4))
        acc = lax.fori_loop(0, lax.div(ln + 1, 2), accum2, acc0)

        den = jnp.full((LANES,), lax.max(ln, 1), jnp.int32).astype(jnp.float32)
        for l in range(4):
            out_v[b, pl.ds(l * LANES, LANES)] = acc[l] / den

    for j in range(NBUF):
        fire(j, j)

    def group(i, _):
        b0 = NBUF * i
        for j in range(NBUF):
            b = b0 + j
            drain_sum(b, j)

            @pl.when(b + NBUF < ROWS_PER_W)
            def _():
                fire(b + NBUF, j)

        return 0

    lax.fori_loop(0, ROWS_PER_W // NBUF, group, 0)

    pltpu.sync_copy(out_v, out_hbm.at[pl.ds(base, ROWS_PER_W), :])


@jax.jit
def _pooled(token_ids, token_lens, table):
    ids_flat = token_ids.reshape(BATCH * SEQ)
    mesh = plsc.VectorSubcoreMesh(core_axis_name="c", subcore_axis_name="s")

    lin = functools.partial(
        pl.kernel,
        mesh=mesh,
        compiler_params=pltpu.CompilerParams(use_tc_tiling_on_sc=True,
                                             needs_layout_passes=False),
        out_type=jax.ShapeDtypeStruct((VOCAB * EMBED_DIM,), jnp.float32),
        scratch_types=[
            pltpu.VMEM((2, VBLK, EMBED_DIM), jnp.float32),
            pltpu.VMEM((2, VBLK * EMBED_DIM), jnp.float32),
            pltpu.SemaphoreType.DMA,
            pltpu.SemaphoreType.DMA,
        ],
    )(_lin_body)
    table_lin = lin(table).reshape(VOCAB, EMBED_DIM)

    gather = functools.partial(
        pl.kernel,
        mesh=mesh,
        compiler_params=pltpu.CompilerParams(use_tc_tiling_on_sc=False,
                                             needs_layout_passes=False),
        out_type=jax.ShapeDtypeStruct((BATCH, EMBED_DIM), jnp.float32),
        scratch_types=[
            pltpu.VMEM((ROWS_PER_W * SEQ,), jnp.int32),
            pltpu.VMEM((ROWS_PER_W + LANES,), jnp.int32),
            pltpu.VMEM((NBUF, SEQ, EMBED_DIM), jnp.float32),
            pltpu.VMEM((ROWS_PER_W, EMBED_DIM), jnp.float32),
            pltpu.SemaphoreType.DMA,
            pltpu.SemaphoreType.DMA,
        ],
    )(_body)
    return gather(ids_flat, token_lens, table_lin)


def kernel(token_ids, token_lens, table):
    return _pooled(token_ids, token_lens, table)


# speedup vs baseline: 1.1139x; 1.1139x over previous
import functools

import jax
import jax.numpy as jnp
from jax import lax
from jax.experimental import pallas as pl
from jax.experimental.pallas import tpu as pltpu
from jax.experimental.pallas import tpu_sc as plsc

BATCH = 4096
SEQ = 200
EMBED_DIM = 64
PAD_DIM = 128
LANES = 16
NUM_WORKERS = 32
ROWS_PER_W = BATCH // NUM_WORKERS
CHUNK = 48
LAST_OFF = SEQ - CHUNK
NBUF = 2


def _body(ids_hbm, lens_hbm, table_hbm, out_hbm, ids_v, lens_v, rows_v,
          out_v, sem0, sem1):
    cid = lax.axis_index("c")
    sid = lax.axis_index("s")
    wid = sid * 2 + cid
    base = wid * ROWS_PER_W
    sems = (sem0, sem1)

    pltpu.sync_copy(ids_hbm.at[pl.ds(base * SEQ, ROWS_PER_W * SEQ)], ids_v)
    pltpu.sync_copy(lens_hbm.at[pl.ds(base, ROWS_PER_W)],
                    lens_v.at[pl.ds(0, ROWS_PER_W)])

    def nchunks(b):
        ln = lens_v[pl.ds(b, LANES)][0]
        return ln, lax.div(ln + (CHUNK - 1), CHUNK)

    def fire(b, buf):
        _, nch = nchunks(b)

        def chunk(c, _):
            off = lax.min(c * CHUNK, LAST_OFF)
            pltpu.async_copy(
                table_hbm.at[ids_v.at[pl.ds(b * SEQ + off, CHUNK)]],
                rows_v.at[buf, pl.ds(off, CHUNK), :],
                sems[buf],
            )
            return 0

        lax.fori_loop(0, nch, chunk, 0)

    def drain_sum(b, buf):
        ln, nch = nchunks(b)

        def dchunk(c, _):
            off = lax.min(c * CHUNK, LAST_OFF)
            pltpu.make_async_copy(
                table_hbm.at[ids_v.at[pl.ds(b * SEQ + off, CHUNK)]],
                rows_v.at[buf, pl.ds(off, CHUNK), :],
                sems[buf],
            ).wait()
            return 0

        lax.fori_loop(0, nch, dchunk, 0)

        def load4(s):
            return [rows_v[buf, s, pl.ds(l * LANES, LANES)]
                    for l in range(4)]

        zero = jnp.zeros((LANES,), jnp.float32)

        def accum2(i, acc):
            r0 = load4(2 * i)
            r1 = load4(2 * i + 1)
            keep = jnp.full((LANES,), 2 * i + 1 < ln)
            return tuple(
                acc[l] + r0[l] + lax.select(keep, r1[l], zero)
                for l in range(4)
            )

        acc0 = tuple(zero for _ in range(4))
        acc = lax.fori_loop(0, lax.div(ln + 1, 2), accum2, acc0)

        den = jnp.full((LANES,), lax.max(ln, 1), jnp.int32).astype(jnp.float32)
        for l in range(4):
            out_v[b, pl.ds(l * LANES, LANES)] = acc[l] / den

    for j in range(NBUF):
        fire(j, j)

    def group(i, _):
        b0 = NBUF * i
        for j in range(NBUF):
            b = b0 + j
            drain_sum(b, j)

            @pl.when(b + NBUF < ROWS_PER_W)
            def _():
                fire(b + NBUF, j)

        return 0

    lax.fori_loop(0, ROWS_PER_W // NBUF, group, 0)

    pltpu.sync_copy(out_v, out_hbm.at[pl.ds(base, ROWS_PER_W), :])


@jax.jit
def _pooled(token_ids, token_lens, table):
    ids_flat = token_ids.reshape(BATCH * SEQ)
    table_pad = jnp.pad(table, ((0, 0), (0, PAD_DIM - EMBED_DIM)))
    mesh = plsc.VectorSubcoreMesh(core_axis_name="c", subcore_axis_name="s")
    f = functools.partial(
        pl.kernel,
        mesh=mesh,
        compiler_params=pltpu.CompilerParams(use_tc_tiling_on_sc=False,
                                             needs_layout_passes=False),
        out_type=jax.ShapeDtypeStruct((BATCH, EMBED_DIM), jnp.float32),
        scratch_types=[
            pltpu.VMEM((ROWS_PER_W * SEQ,), jnp.int32),
            pltpu.VMEM((ROWS_PER_W + LANES,), jnp.int32),
            pltpu.VMEM((NBUF, SEQ, PAD_DIM), jnp.float32),
            pltpu.VMEM((ROWS_PER_W, EMBED_DIM), jnp.float32),
            pltpu.SemaphoreType.DMA,
            pltpu.SemaphoreType.DMA,
        ],
    )(_body)
    return f(ids_flat, token_lens, table_pad)


def kernel(token_ids, token_lens, table):
    return _pooled(token_ids, token_lens, table)


# speedup vs baseline: 1.1912x; 1.0694x over previous
import functools

import jax
import jax.numpy as jnp
from jax import lax
from jax.experimental import pallas as pl
from jax.experimental.pallas import tpu as pltpu
from jax.experimental.pallas import tpu_sc as plsc

BATCH = 4096
SEQ = 200
EMBED_DIM = 64
PAD_DIM = 128
LANES = 16
NUM_WORKERS = 32
ROWS_PER_W = BATCH // NUM_WORKERS
CHUNK = 48
LAST_OFF = SEQ - CHUNK
NBUF = 2


def _body(ids_hbm, lens_hbm, table_hbm, out_hbm, ids_v, lens_v, rows_v,
          out_v, sem0, sem1):
    cid = lax.axis_index("c")
    sid = lax.axis_index("s")
    wid = sid * 2 + cid
    base = wid * ROWS_PER_W
    sems = (sem0, sem1)

    pltpu.sync_copy(ids_hbm.at[pl.ds(base * SEQ, ROWS_PER_W * SEQ)], ids_v)
    pltpu.sync_copy(lens_hbm.at[pl.ds(base, ROWS_PER_W)],
                    lens_v.at[pl.ds(0, ROWS_PER_W)])

    def dbl(r, _):
        for u in range(4):
            off = (4 * r + u) * LANES
            ids_v[pl.ds(off, LANES)] = ids_v[pl.ds(off, LANES)] * 2
        return 0

    lax.fori_loop(0, ROWS_PER_W * SEQ // (4 * LANES), dbl, 0)

    def nchunks(b):
        ln = lens_v[pl.ds(b, LANES)][0]
        return ln, lax.div(ln + (CHUNK - 1), CHUNK)

    def fire(b, buf):
        _, nch = nchunks(b)

        def chunk(c, _):
            off = lax.min(c * CHUNK, LAST_OFF)
            pltpu.async_copy(
                table_hbm.at[ids_v.at[pl.ds(b * SEQ + off, CHUNK)]],
                rows_v.at[buf, pl.ds(off, CHUNK), :],
                sems[buf],
            )
            return 0

        lax.fori_loop(0, nch, chunk, 0)

    def drain_sum(b, buf):
        ln, nch = nchunks(b)

        def dchunk(c, _):
            off = lax.min(c * CHUNK, LAST_OFF)
            pltpu.make_async_copy(
                table_hbm.at[ids_v.at[pl.ds(b * SEQ + off, CHUNK)]],
                rows_v.at[buf, pl.ds(off, CHUNK), :],
                sems[buf],
            ).wait()
            return 0

        lax.fori_loop(0, nch, dchunk, 0)

        def load4(s):
            return [rows_v[buf, s, pl.ds(l * LANES, LANES)]
                    for l in range(4)]

        zero = jnp.zeros((LANES,), jnp.float32)

        def accum2(i, acc):
            r0 = load4(2 * i)
            r1 = load4(2 * i + 1)
            keep = jnp.full((LANES,), 2 * i + 1 < ln)
            return tuple(
                acc[l] + r0[l] + lax.select(keep, r1[l], zero)
                for l in range(4)
            )

        acc0 = tuple(zero for _ in range(4))
        acc = lax.fori_loop(0, lax.div(ln + 1, 2), accum2, acc0)

        den = jnp.full((LANES,), lax.max(ln, 1), jnp.int32).astype(jnp.float32)
        for l in range(4):
            out_v[b, pl.ds(l * LANES, LANES)] = acc[l] / den

    for j in range(NBUF):
        fire(j, j)

    def group(i, _):
        b0 = NBUF * i
        for j in range(NBUF):
            b = b0 + j
            drain_sum(b, j)

            @pl.when(b + NBUF < ROWS_PER_W)
            def _():
                fire(b + NBUF, j)

        return 0

    lax.fori_loop(0, ROWS_PER_W // NBUF, group, 0)

    pltpu.sync_copy(out_v, out_hbm.at[pl.ds(base, ROWS_PER_W), :])


@jax.jit
def _pooled(token_ids, token_lens, table):
    ids_flat = token_ids.reshape(BATCH * SEQ)
    table_pad = jnp.pad(table, ((0, 0), (0, PAD_DIM - EMBED_DIM)))
    table_pairs = table_pad.reshape(2 * 1000000, EMBED_DIM)
    mesh = plsc.VectorSubcoreMesh(core_axis_name="c", subcore_axis_name="s")
    f = functools.partial(
        pl.kernel,
        mesh=mesh,
        compiler_params=pltpu.CompilerParams(use_tc_tiling_on_sc=False,
                                             needs_layout_passes=False),
        out_type=jax.ShapeDtypeStruct((BATCH, EMBED_DIM), jnp.float32),
        scratch_types=[
            pltpu.VMEM((ROWS_PER_W * SEQ,), jnp.int32),
            pltpu.VMEM((ROWS_PER_W + LANES,), jnp.int32),
            pltpu.VMEM((NBUF, SEQ, EMBED_DIM), jnp.float32),
            pltpu.VMEM((ROWS_PER_W, EMBED_DIM), jnp.float32),
            pltpu.SemaphoreType.DMA,
            pltpu.SemaphoreType.DMA,
        ],
    )(_body)
    return f(ids_flat, token_lens, table_pairs)


def kernel(token_ids, token_lens, table):
    return _pooled(token_ids, token_lens, table)


# speedup vs baseline: 1.2493x; 1.0488x over previous
import functools

import jax
import jax.numpy as jnp
from jax import lax
from jax.experimental import pallas as pl
from jax.experimental.pallas import tpu as pltpu
from jax.experimental.pallas import tpu_sc as plsc

BATCH = 4096
SEQ = 200
EMBED_DIM = 64
PAD_DIM = 128
LANES = 16
NUM_WORKERS = 32
ROWS_PER_W = BATCH // NUM_WORKERS
CHUNK = 48
LAST_OFF = SEQ - CHUNK
NBUF = 4


def _body(ids_hbm, lens_hbm, table_hbm, out_hbm, ids_v, lens_v, rows_v,
          out_v, sem0, sem1, sem2, sem3):
    cid = lax.axis_index("c")
    sid = lax.axis_index("s")
    wid = sid * 2 + cid
    base = wid * ROWS_PER_W
    sems = (sem0, sem1, sem2, sem3)

    pltpu.sync_copy(ids_hbm.at[pl.ds(base * SEQ, ROWS_PER_W * SEQ)], ids_v)
    pltpu.sync_copy(lens_hbm.at[pl.ds(base, ROWS_PER_W)],
                    lens_v.at[pl.ds(0, ROWS_PER_W)])

    def dbl(r, _):
        for u in range(4):
            off = (4 * r + u) * LANES
            ids_v[pl.ds(off, LANES)] = ids_v[pl.ds(off, LANES)] * 2
        return 0

    lax.fori_loop(0, ROWS_PER_W * SEQ // (4 * LANES), dbl, 0)

    def nchunks(b):
        ln = lens_v[pl.ds(b, LANES)][0]
        return ln, lax.div(ln + (CHUNK - 1), CHUNK)

    def fire(b, buf):
        _, nch = nchunks(b)

        def chunk(c, _):
            off = lax.min(c * CHUNK, LAST_OFF)
            pltpu.async_copy(
                table_hbm.at[ids_v.at[pl.ds(b * SEQ + off, CHUNK)]],
                rows_v.at[buf, pl.ds(off, CHUNK), :],
                sems[buf],
            )
            return 0

        lax.fori_loop(0, nch, chunk, 0)

    def drain_sum(b, buf):
        ln, nch = nchunks(b)

        def dchunk(c, _):
            off = lax.min(c * CHUNK, LAST_OFF)
            pltpu.make_async_copy(
                table_hbm.at[ids_v.at[pl.ds(b * SEQ + off, CHUNK)]],
                rows_v.at[buf, pl.ds(off, CHUNK), :],
                sems[buf],
            ).wait()
            return 0

        lax.fori_loop(0, nch, dchunk, 0)

        def load4(s):
            return [rows_v[buf, s, pl.ds(l * LANES, LANES)]
                    for l in range(4)]

        zero = jnp.zeros((LANES,), jnp.float32)

        def accum2(i, acc):
            r0 = load4(2 * i)
            r1 = load4(2 * i + 1)
            keep = jnp.full((LANES,), 2 * i + 1 < ln)
            return tuple(
                acc[l] + r0[l] + lax.select(keep, r1[l], zero)
                for l in range(4)
            )

        acc0 = tuple(zero for _ in range(4))
        acc = lax.fori_loop(0, lax.div(ln + 1, 2), accum2, acc0)

        den = jnp.full((LANES,), lax.max(ln, 1), jnp.int32).astype(jnp.float32)
        for l in range(4):
            out_v[b, pl.ds(l * LANES, LANES)] = acc[l] / den

    for j in range(NBUF):
        fire(j, j)

    def group(i, _):
        b0 = NBUF * i
        for j in range(NBUF):
            b = b0 + j
            drain_sum(b, j)

            @pl.when(b + NBUF < ROWS_PER_W)
            def _():
                fire(b + NBUF, j)

        return 0

    lax.fori_loop(0, ROWS_PER_W // NBUF, group, 0)

    pltpu.sync_copy(out_v, out_hbm.at[pl.ds(base, ROWS_PER_W), :])


@jax.jit
def _pooled(token_ids, token_lens, table):
    ids_flat = token_ids.reshape(BATCH * SEQ)
    table_pad = jnp.pad(table, ((0, 0), (0, PAD_DIM - EMBED_DIM)))
    table_pairs = table_pad.reshape(2 * 1000000, EMBED_DIM)
    mesh = plsc.VectorSubcoreMesh(core_axis_name="c", subcore_axis_name="s")
    f = functools.partial(
        pl.kernel,
        mesh=mesh,
        compiler_params=pltpu.CompilerParams(use_tc_tiling_on_sc=False,
                                             needs_layout_passes=False),
        out_type=jax.ShapeDtypeStruct((BATCH, EMBED_DIM), jnp.float32),
        scratch_types=[
            pltpu.VMEM((ROWS_PER_W * SEQ,), jnp.int32),
            pltpu.VMEM((ROWS_PER_W + LANES,), jnp.int32),
            pltpu.VMEM((NBUF, SEQ, EMBED_DIM), jnp.float32),
            pltpu.VMEM((ROWS_PER_W, EMBED_DIM), jnp.float32),
            pltpu.SemaphoreType.DMA,
            pltpu.SemaphoreType.DMA,
            pltpu.SemaphoreType.DMA,
            pltpu.SemaphoreType.DMA,
        ],
    )(_body)
    return f(ids_flat, token_lens, table_pairs)


def kernel(token_ids, token_lens, table):
    return _pooled(token_ids, token_lens, table)
